# baseline (speedup 1.0000x reference)
import functools

import jax
import jax.numpy as jnp
from jax import lax
from jax.experimental import pallas as pl
from jax.experimental.pallas import tpu as pltpu
from jax.experimental.pallas import tpu_sc as plsc

_B = 1024
_H = 19
_W = 19
_HW = _H * _W
_F = 64
_V = 2380
_L = 16
_NFULL = _HW // _L
_TOFF = _NFULL * _L
_NTAIL = _HW - _TOFF
_NC = 2
_NS = 16
_BPT = _B // _NS
_FH = _F // _NC
_FP = _FH // 2
_OC = 3 + _F
_VR = _V + 1
_ZROW = _V
_TWORDS = _VR * _FP


def _splat(v):
    return jnp.full((_L,), v, jnp.int32)


def _unpack_pair(g):
    lo = plsc.bitcast(g << 16, jnp.float32)
    hi = plsc.bitcast(g & jnp.int32(-65536), jnp.float32)
    return lo, hi


def _sc_body(brdp_hbm, stm_hbm, idxp_hbm, tbl2_hbm, out_hbm,
             table_v, outb_v, idx_v, brd_v, stm_v, sem_out0, sem_out1):
    cid = lax.axis_index("c")
    sid = lax.axis_index("s")
    bbase = sid * _BPT
    sem_out = (sem_out0, sem_out1)

    pltpu.sync_copy(tbl2_hbm.at[cid], table_v)
    pltpu.sync_copy(idxp_hbm.at[pl.ds(bbase, _BPT)], idx_v)
    pltpu.sync_copy(brdp_hbm.at[pl.ds(bbase, _BPT)], brd_v)
    pltpu.sync_copy(stm_hbm.at[pl.ds(bbase, _BPT)], stm_v)

    mask16 = jnp.int32(0xFFFF)
    hi_mask = jnp.int32(-65536)
    zsplat = _splat(_ZROW)

    def out_copy(b, t):
        return (
            pltpu.make_async_copy(outb_v.at[t],
                                  out_hbm.at[b, pl.ds(0, 3 + _FH), :],
                                  sem_out[t]),
            pltpu.make_async_copy(outb_v.at[t, pl.ds(3, _FH), :],
                                  out_hbm.at[b, pl.ds(3 + _FH, _FH), :],
                                  sem_out[t]),
        )

    def compute_sample(s, t):
        b = bbase + s
        stm16 = plsc.load_gather(stm_v, [_splat(0) + s])

        c0, c1 = out_copy(b, t)

        @pl.when((s >= 2) & (cid == 0))
        def _():
            c0.wait()

        @pl.when((s >= 2) & (cid == 1))
        def _():
            c1.wait()

        def _chunk(c, carry_c):
            off = jnp.minimum(c * _L, _HW - _L)
            w_i = idx_v[s, pl.ds(off, _L)]
            w_b = brd_v[s, pl.ds(off, _L)]
            occ = w_b != 0
            base0 = jnp.where(occ, zsplat, w_i & mask16)
            base1 = jnp.where(occ, zsplat,
                              lax.shift_right_logical(w_i, 16))

            gs = []
            for f2 in range(_FP):
                gs.append(plsc.load_gather(table_v, [base0 + f2 * _VR]))
                gs.append(plsc.load_gather(table_v, [base1 + f2 * _VR]))
            outs = []
            for f2 in range(_FP):
                lo0, hi0 = _unpack_pair(gs[2 * f2])
                lo1, hi1 = _unpack_pair(gs[2 * f2 + 1])
                outs.append(lo0 + lo1)
                outs.append(hi0 + hi1)

            @pl.when(cid == 0)
            def _():
                b0, b1 = _unpack_pair(w_b)
                outb_v[t, 0, pl.ds(off, _L)] = b0
                outb_v[t, 1, pl.ds(off, _L)] = b1
                outb_v[t, 2, pl.ds(off, _L)] = stm16

            for f in range(_FH):
                outb_v[t, 3 + f, pl.ds(off, _L)] = outs[f]
            return carry_c

        lax.fori_loop(0, _NFULL + 1, _chunk, 0)

        plsc.subcore_barrier()

        c0, c1 = out_copy(b, t)

        @pl.when(cid == 0)
        def _():
            c0.start()

        @pl.when(cid == 1)
        def _():
            c1.start()

    def sample_pair(j, carry):
        compute_sample(j * 2, 0)
        compute_sample(j * 2 + 1, 1)
        return carry

    lax.fori_loop(0, _BPT // 2, sample_pair, 0)

    for t in range(2):
        c0, c1 = out_copy(bbase + _BPT - 2 + t, t)

        @pl.when(cid == 0)
        def _():
            c0.wait()

        @pl.when(cid == 1)
        def _():
            c1.wait()


@jax.jit
def _sc_call(brd_packed, stm, idx_packed, tbl2):
    mesh = plsc.VectorSubcoreMesh(core_axis_name="c", subcore_axis_name="s",
                                  num_cores=_NC, num_subcores=_NS)
    return pl.kernel(
        _sc_body,
        out_type=jax.ShapeDtypeStruct((_B, _OC, _HW), jnp.float32),
        mesh=mesh,
        compiler_params=pltpu.CompilerParams(use_tc_tiling_on_sc=False,
                                             needs_layout_passes=False,
                                             disable_bounds_checks=True),
        scratch_types=[
            pltpu.VMEM((_TWORDS,), jnp.int32),
            pltpu.VMEM((2, 3 + _FH, _HW), jnp.float32),
            pltpu.VMEM((_BPT, _HW), jnp.int32),
            pltpu.VMEM((_BPT, _HW), jnp.int32),
            pltpu.VMEM((_BPT,), jnp.float32),
            pltpu.SemaphoreType.DMA,
            pltpu.SemaphoreType.DMA,
        ],
    )(brd_packed, stm, idx_packed, tbl2)


def kernel(board_input, stm_input, sparse_feature_input, sparse_feature_dim,
           pcode_embedding):
    del sparse_feature_dim
    sparse3 = sparse_feature_input.reshape(_B, 12, _HW)
    idx_packed = sparse3[:, 10, :] | (sparse3[:, 11, :] << 16)
    brd_packed = jax.lax.bitcast_convert_type(
        board_input.reshape(_B, 2, _HW).transpose(0, 2, 1)
        .astype(jnp.bfloat16), jnp.int32)
    tbl = jnp.concatenate(
        [pcode_embedding, jnp.zeros((1, _F), jnp.float32)], axis=0)
    tbl_pairs = jax.lax.bitcast_convert_type(
        tbl.astype(jnp.bfloat16).reshape(_V + 1, _F // 2, 2), jnp.int32)
    tbl2 = tbl_pairs.reshape(_VR, _NC, _FP).transpose(1, 2, 0) \
        .reshape(_NC, _TWORDS)
    out = _sc_call(brd_packed, stm_input, idx_packed, tbl2)
    return out.reshape(_B, _OC, _H, _W)


# speedup vs baseline: 1.0044x; 1.0044x over previous
import functools

import jax
import jax.numpy as jnp
from jax import lax
from jax.experimental import pallas as pl
from jax.experimental.pallas import tpu as pltpu
from jax.experimental.pallas import tpu_sc as plsc

_B = 1024
_H = 19
_W = 19
_HW = _H * _W
_F = 64
_V = 2380
_L = 16
_NFULL = _HW // _L
_TOFF = _NFULL * _L
_NTAIL = _HW - _TOFF
_NC = 2
_NS = 16
_BPT = _B // _NS
_FH = _F // _NC
_FP = _FH // 2
_OC = 3 + _F
_VR = _V + 1
_ZROW = _V
_TWORDS = _VR * _FP


def _splat(v):
    return jnp.full((_L,), v, jnp.int32)


def _unpack_pair(g):
    lo = plsc.bitcast(g << 16, jnp.float32)
    hi = plsc.bitcast(g & jnp.int32(-65536), jnp.float32)
    return lo, hi


def _sc_body(brdp_hbm, stm_hbm, idxp_hbm, tbl2_hbm, out_hbm,
             table_v, outb_v, idx_v, brd_v, stm_v, sem_out0, sem_out1):
    cid = lax.axis_index("c")
    sid = lax.axis_index("s")
    bbase = sid * _BPT
    sem_out = (sem_out0, sem_out1)

    pltpu.sync_copy(tbl2_hbm.at[cid], table_v)
    pltpu.sync_copy(idxp_hbm.at[pl.ds(bbase, _BPT)], idx_v)
    pltpu.sync_copy(brdp_hbm.at[pl.ds(bbase, _BPT)], brd_v)
    pltpu.sync_copy(stm_hbm.at[pl.ds(bbase, _BPT)], stm_v)

    mask16 = jnp.int32(0xFFFF)
    hi_mask = jnp.int32(-65536)
    zsplat = _splat(_ZROW)

    def out_copy(b, t):
        return (
            pltpu.make_async_copy(outb_v.at[t],
                                  out_hbm.at[b, pl.ds(0, 3 + _FH), :],
                                  sem_out[t]),
            pltpu.make_async_copy(outb_v.at[t, pl.ds(3, _FH), :],
                                  out_hbm.at[b, pl.ds(3 + _FH, _FH), :],
                                  sem_out[t]),
        )

    def compute_sample(s, t):
        b = bbase + s
        stm16 = plsc.load_gather(stm_v, [_splat(0) + s])

        c0, c1 = out_copy(b, t)

        @pl.when((s >= 2) & (cid == 0))
        def _():
            c0.wait()

        @pl.when((s >= 2) & (cid == 1))
        def _():
            c1.wait()

        def _emit_chunk(off):
            w_i = idx_v[s, pl.ds(off, _L)]
            w_b = brd_v[s, pl.ds(off, _L)]
            occ = w_b != 0
            base0 = jnp.where(occ, zsplat, w_i & mask16)
            base1 = jnp.where(occ, zsplat,
                              lax.shift_right_logical(w_i, 16))

            gs = []
            for f2 in range(_FP):
                gs.append(plsc.load_gather(table_v, [base0 + f2 * _VR]))
                gs.append(plsc.load_gather(table_v, [base1 + f2 * _VR]))
            outs = []
            for f2 in range(_FP):
                lo0, hi0 = _unpack_pair(gs[2 * f2])
                lo1, hi1 = _unpack_pair(gs[2 * f2 + 1])
                outs.append(lo0 + lo1)
                outs.append(hi0 + hi1)

            @pl.when(cid == 0)
            def _():
                b0, b1 = _unpack_pair(w_b)
                outb_v[t, 0, pl.ds(off, _L)] = b0
                outb_v[t, 1, pl.ds(off, _L)] = b1
                outb_v[t, 2, pl.ds(off, _L)] = stm16

            for f in range(_FH):
                outb_v[t, 3 + f, pl.ds(off, _L)] = outs[f]

        def _chunk2(c, carry_c):
            _emit_chunk(c * (2 * _L))
            _emit_chunk(c * (2 * _L) + _L)
            return carry_c

        lax.fori_loop(0, _NFULL // 2, _chunk2, 0)
        _emit_chunk(jnp.int32(_HW - _L))

        plsc.subcore_barrier()

        c0, c1 = out_copy(b, t)

        @pl.when(cid == 0)
        def _():
            c0.start()

        @pl.when(cid == 1)
        def _():
            c1.start()

    def sample_pair(j, carry):
        compute_sample(j * 2, 0)
        compute_sample(j * 2 + 1, 1)
        return carry

    lax.fori_loop(0, _BPT // 2, sample_pair, 0)

    for t in range(2):
        c0, c1 = out_copy(bbase + _BPT - 2 + t, t)

        @pl.when(cid == 0)
        def _():
            c0.wait()

        @pl.when(cid == 1)
        def _():
            c1.wait()


@jax.jit
def _sc_call(brd_packed, stm, idx_packed, tbl2):
    mesh = plsc.VectorSubcoreMesh(core_axis_name="c", subcore_axis_name="s",
                                  num_cores=_NC, num_subcores=_NS)
    return pl.kernel(
        _sc_body,
        out_type=jax.ShapeDtypeStruct((_B, _OC, _HW), jnp.float32),
        mesh=mesh,
        compiler_params=pltpu.CompilerParams(use_tc_tiling_on_sc=False,
                                             needs_layout_passes=False,
                                             disable_bounds_checks=True),
        scratch_types=[
            pltpu.VMEM((_TWORDS,), jnp.int32),
            pltpu.VMEM((2, 3 + _FH, _HW), jnp.float32),
            pltpu.VMEM((_BPT, _HW), jnp.int32),
            pltpu.VMEM((_BPT, _HW), jnp.int32),
            pltpu.VMEM((_BPT,), jnp.float32),
            pltpu.SemaphoreType.DMA,
            pltpu.SemaphoreType.DMA,
        ],
    )(brd_packed, stm, idx_packed, tbl2)


def kernel(board_input, stm_input, sparse_feature_input, sparse_feature_dim,
           pcode_embedding):
    del sparse_feature_dim
    sparse3 = sparse_feature_input.reshape(_B, 12, _HW)
    idx_packed = sparse3[:, 10, :] | (sparse3[:, 11, :] << 16)
    brd_packed = jax.lax.bitcast_convert_type(
        board_input.reshape(_B, 2, _HW).transpose(0, 2, 1)
        .astype(jnp.bfloat16), jnp.int32)
    tbl = jnp.concatenate(
        [pcode_embedding, jnp.zeros((1, _F), jnp.float32)], axis=0)
    tbl_pairs = jax.lax.bitcast_convert_type(
        tbl.astype(jnp.bfloat16).reshape(_V + 1, _F // 2, 2), jnp.int32)
    tbl2 = tbl_pairs.reshape(_VR, _NC, _FP).transpose(1, 2, 0) \
        .reshape(_NC, _TWORDS)
    out = _sc_call(brd_packed, stm_input, idx_packed, tbl2)
    return out.reshape(_B, _OC, _H, _W)


# speedup vs baseline: 1.0045x; 1.0001x over previous
import jax
import jax.numpy as jnp
from jax import lax
from jax.experimental import pallas as pl
from jax.experimental.pallas import tpu as pltpu
from jax.experimental.pallas import tpu_sc as plsc

_B = 1024
_H = 19
_W = 19
_HW = _H * _W
_F = 64
_V = 2380
_L = 16
_NFULL = _HW // _L
_NC = 2
_NS = 16
_BPT = _B // _NS
_FH = _F // _NC
_FP = _FH // 2
_OC = 3 + _F
_VR = _V + 1
_ZROW = _V
_TWORDS = _VR * _FP


def _splat(v):
    return jnp.full((_L,), v, jnp.int32)


def _unpack_pair(g):
    lo = plsc.bitcast(g << 16, jnp.float32)
    hi = plsc.bitcast(g & jnp.int32(-65536), jnp.float32)
    return lo, hi


def _sc_body(brdp_hbm, stm_hbm, idxp_hbm, tbl2_hbm, out_hbm,
             table_v, outb_v, idx_v, brd_v, stm_v, sem_out0, sem_out1):
    cid = lax.axis_index("c")
    sid = lax.axis_index("s")
    bbase = sid * _BPT
    sem_out = (sem_out0, sem_out1)

    pltpu.sync_copy(tbl2_hbm.at[cid], table_v)
    pltpu.sync_copy(idxp_hbm.at[pl.ds(bbase, _BPT)], idx_v)
    pltpu.sync_copy(brdp_hbm.at[pl.ds(bbase, _BPT)], brd_v)
    pltpu.sync_copy(stm_hbm.at[pl.ds(bbase, _BPT)], stm_v)

    mask16 = jnp.int32(0xFFFF)
    hi_mask = jnp.int32(-65536)
    zsplat = _splat(_ZROW)

    def out_copy(b, t):
        return (
            pltpu.make_async_copy(outb_v.at[t],
                                  out_hbm.at[b, pl.ds(0, 3 + _FH), :],
                                  sem_out[t]),
            pltpu.make_async_copy(outb_v.at[t, pl.ds(3, _FH), :],
                                  out_hbm.at[b, pl.ds(3 + _FH, _FH), :],
                                  sem_out[t]),
        )

    def compute_sample(s, t):
        b = bbase + s
        stm16 = plsc.load_gather(stm_v, [_splat(0) + s])

        c0, c1 = out_copy(b, t)

        @pl.when((s >= 2) & (cid == 0))
        def _():
            c0.wait()

        @pl.when((s >= 2) & (cid == 1))
        def _():
            c1.wait()

        def _emit_chunk(off):
            w_i = idx_v[s, pl.ds(off, _L)]
            w_b = brd_v[s, pl.ds(off, _L)]
            occ = w_b != 0
            base0 = jnp.where(occ, zsplat, w_i & mask16)
            base1 = jnp.where(occ, zsplat,
                              lax.shift_right_logical(w_i, 16))

            gs = []
            for f2 in range(_FP):
                gs.append(plsc.load_gather(table_v, [base0 + f2 * _VR]))
                gs.append(plsc.load_gather(table_v, [base1 + f2 * _VR]))
            outs = []
            for f2 in range(_FP):
                lo0, hi0 = _unpack_pair(gs[2 * f2])
                lo1, hi1 = _unpack_pair(gs[2 * f2 + 1])
                outs.append(lo0 + lo1)
                outs.append(hi0 + hi1)

            @pl.when(cid == 0)
            def _():
                b0, b1 = _unpack_pair(w_b)
                outb_v[t, 0, pl.ds(off, _L)] = b0
                outb_v[t, 1, pl.ds(off, _L)] = b1
                outb_v[t, 2, pl.ds(off, _L)] = stm16

            for f in range(_FH):
                outb_v[t, 3 + f, pl.ds(off, _L)] = outs[f]

        def _chunk2(c, carry_c):
            _emit_chunk(c * (2 * _L))
            _emit_chunk(c * (2 * _L) + _L)
            return carry_c

        lax.fori_loop(0, _NFULL // 2, _chunk2, 0)
        _emit_chunk(jnp.int32(_HW - _L))

        plsc.subcore_barrier()

        c0, c1 = out_copy(b, t)

        @pl.when(cid == 0)
        def _():
            c0.start()

        @pl.when(cid == 1)
        def _():
            c1.start()

    def sample_pair(j, carry):
        compute_sample(j * 2, 0)
        compute_sample(j * 2 + 1, 1)
        return carry

    lax.fori_loop(0, _BPT // 2, sample_pair, 0)

    for t in range(2):
        c0, c1 = out_copy(bbase + _BPT - 2 + t, t)

        @pl.when(cid == 0)
        def _():
            c0.wait()

        @pl.when(cid == 1)
        def _():
            c1.wait()


@jax.jit
def _sc_call(brd_packed, stm, idx_packed, tbl2):
    mesh = plsc.VectorSubcoreMesh(core_axis_name="c", subcore_axis_name="s",
                                  num_cores=_NC, num_subcores=_NS)
    return pl.kernel(
        _sc_body,
        out_type=jax.ShapeDtypeStruct((_B, _OC, _HW), jnp.float32),
        mesh=mesh,
        compiler_params=pltpu.CompilerParams(use_tc_tiling_on_sc=False,
                                             needs_layout_passes=False,
                                             disable_bounds_checks=True),
        scratch_types=[
            pltpu.VMEM((_TWORDS,), jnp.int32),
            pltpu.VMEM((2, 3 + _FH, _HW), jnp.float32),
            pltpu.VMEM((_BPT, _HW), jnp.int32),
            pltpu.VMEM((_BPT, _HW), jnp.int32),
            pltpu.VMEM((_BPT,), jnp.float32),
            pltpu.SemaphoreType.DMA,
            pltpu.SemaphoreType.DMA,
        ],
    )(brd_packed, stm, idx_packed, tbl2)


def kernel(board_input, stm_input, sparse_feature_input, sparse_feature_dim,
           pcode_embedding):
    del sparse_feature_dim
    sparse3 = sparse_feature_input.reshape(_B, 12, _HW)
    idx_packed = sparse3[:, 10, :] | (sparse3[:, 11, :] << 16)
    brd_packed = jax.lax.bitcast_convert_type(
        board_input.reshape(_B, 2, _HW).transpose(0, 2, 1)
        .astype(jnp.bfloat16), jnp.int32)
    tbl = jnp.concatenate(
        [pcode_embedding, jnp.zeros((1, _F), jnp.float32)], axis=0)
    tbl_pairs = jax.lax.bitcast_convert_type(
        tbl.astype(jnp.bfloat16).reshape(_V + 1, _F // 2, 2), jnp.int32)
    tbl2 = tbl_pairs.reshape(_VR, _NC, _FP).transpose(1, 2, 0) \
        .reshape(_NC, _TWORDS)
    out = _sc_call(brd_packed, stm_input, idx_packed, tbl2)
    return out.reshape(_B, _OC, _H, _W)
